# baseline (speedup 1.0000x reference)
import functools

import jax
import jax.numpy as jnp
from jax import lax
from jax.experimental import pallas as pl
from jax.experimental.pallas import tpu as pltpu
from jax.experimental.pallas import tpu_sc as plsc


def _sc_gather_t(user_id, item_id, category, wuT, wiT, wcT):
    info = plsc.get_sparse_core_info()
    NC, NS = info.num_cores, info.num_subcores
    NW = NC * NS
    B = user_id.shape[0]
    DU, VU = wuT.shape
    DC, VC = wcT.shape
    assert DU == 2 * NW and DC == NW
    UNROLL = 8
    n_iter = B // (16 * UNROLL)
    mesh = plsc.VectorSubcoreMesh(core_axis_name="c", subcore_axis_name="s")

    @functools.partial(
        pl.kernel,
        mesh=mesh,
        compiler_params=pltpu.CompilerParams(needs_layout_passes=False),
        out_type=(
            jax.ShapeDtypeStruct((DU, B), jnp.float32),
            jax.ShapeDtypeStruct((DU, B), jnp.float32),
            jax.ShapeDtypeStruct((DC, B), jnp.float32),
        ),
        scratch_types=[
            pltpu.VMEM((B,), jnp.int32),
            pltpu.VMEM((B,), jnp.int32),
            pltpu.VMEM((B,), jnp.int32),
            pltpu.VMEM((VU,), jnp.float32),
            pltpu.VMEM((VC,), jnp.float32),
            pltpu.VMEM((B,), jnp.float32),
            pltpu.VMEM((B,), jnp.float32),
            pltpu.SemaphoreType.DMA,
            pltpu.SemaphoreType.DMA,
            pltpu.SemaphoreType.DMA,
        ],
    )
    def k(uid_h, iid_h, cid_h, wu_h, wi_h, wc_h, eu_h, ei_h, ec_h,
          uidx, iidx, cidx, rowbuf, catbuf, outA, outB,
          sem_row, sem_out, sem_pre):
        w = lax.axis_index("s") * NC + lax.axis_index("c")
        tasks = [(wu_h, w, uidx, eu_h), (wu_h, w + NW, uidx, eu_h),
                 (wi_h, w, iidx, ei_h), (wi_h, w + NW, iidx, ei_h)]
        outs = [outA, outB]

        row_copy = pltpu.async_copy(wu_h.at[w], rowbuf, sem_row)
        pre = [pltpu.async_copy(uid_h, uidx, sem_pre),
               pltpu.async_copy(iid_h, iidx, sem_pre),
               pltpu.async_copy(cid_h, cidx, sem_pre),
               pltpu.async_copy(wc_h.at[w], catbuf, sem_pre)]
        for c in pre:
            c.wait()

        def gather(idxbuf, buf, ob):
            def body(i, carry):
                for u in range(UNROLL):
                    off = (i * UNROLL + u) * 16
                    iv = idxbuf[pl.ds(off, 16)]
                    ob[pl.ds(off, 16)] = plsc.load_gather(buf, [iv])
                return carry

            lax.fori_loop(0, n_iter, body, 0)

        gather(cidx, catbuf, outB)
        out_copies = {-1: pltpu.async_copy(outB, ec_h.at[w], sem_out)}
        for t in range(4):
            row_copy.wait()
            ob = outs[t % 2]
            if t - 2 in out_copies:
                out_copies.pop(t - 2).wait()
            gather(tasks[t][2], rowbuf, ob)
            if t < 3:
                tbl2, row2, _, _ = tasks[t + 1]
                row_copy = pltpu.async_copy(tbl2.at[row2], rowbuf, sem_row)
            out_copies[t] = pltpu.async_copy(
                ob, tasks[t][3].at[tasks[t][1]], sem_out)
        out_copies.pop(2).wait()
        out_copies.pop(3).wait()

    return k(user_id, item_id, category, wuT, wiT, wcT)


def _tc_project(base, euT, eiT, ecT, W_proj, b_proj):
    B, DB = base.shape
    DU = euT.shape[0]
    DI = eiT.shape[0]
    DC = ecT.shape[0]
    N = W_proj.shape[1]
    K = W_proj.shape[0]
    BLK = 4096
    grid = (B // BLK,)
    dn_t = (((0,), (0,)), ((), ()))

    def body(base_ref, eu_ref, ei_ref, ec_ref, wp_ref, b_ref, out_ref):
        acc = jnp.dot(base_ref[...], wp_ref[0:DB, :],
                      preferred_element_type=jnp.float32)
        acc += lax.dot_general(eu_ref[...], wp_ref[DB:DB + DU, :], dn_t,
                               preferred_element_type=jnp.float32)
        acc += lax.dot_general(ei_ref[...], wp_ref[DB + DU:DB + DU + DI, :],
                               dn_t, preferred_element_type=jnp.float32)
        acc += lax.dot_general(ec_ref[...], wp_ref[DB + DU + DI:K, :], dn_t,
                               preferred_element_type=jnp.float32)
        out_ref[...] = acc + b_ref[...]

    return pl.pallas_call(
        body,
        grid=grid,
        in_specs=[
            pl.BlockSpec((BLK, DB), lambda i: (i, 0)),
            pl.BlockSpec((DU, BLK), lambda i: (0, i)),
            pl.BlockSpec((DI, BLK), lambda i: (0, i)),
            pl.BlockSpec((DC, BLK), lambda i: (0, i)),
            pl.BlockSpec((K, N), lambda i: (0, 0)),
            pl.BlockSpec((1, N), lambda i: (0, 0)),
        ],
        out_specs=pl.BlockSpec((BLK, N), lambda i: (i, 0)),
        out_shape=jax.ShapeDtypeStruct((B, N), jnp.float32),
    )(base, euT, eiT, ecT, W_proj, b_proj.reshape(1, N))


def kernel(base, user_id, item_id, category, W_user, W_item, W_cat, W_proj, b_proj):
    user_id = user_id.astype(jnp.int32)
    item_id = item_id.astype(jnp.int32)
    category = category.astype(jnp.int32)
    euT, eiT, ecT = _sc_gather_t(user_id, item_id, category,
                                 W_user.T, W_item.T, W_cat.T)
    return _tc_project(base, euT, eiT, ecT, W_proj, b_proj)


# speedup vs baseline: 1.0157x; 1.0157x over previous
import functools

import jax
import jax.numpy as jnp
from jax import lax
from jax.experimental import pallas as pl
from jax.experimental.pallas import tpu as pltpu
from jax.experimental.pallas import tpu_sc as plsc


def _sc_gather_t(user_id, item_id, category, wuT, wiT, wcT):
    info = plsc.get_sparse_core_info()
    NC, NS = info.num_cores, info.num_subcores
    NW = NC * NS
    B = user_id.shape[0]
    DU, VU = wuT.shape
    DC, VC = wcT.shape
    assert DU == 2 * NW and DC == NW
    UNROLL = 8
    n_iter = B // (16 * UNROLL)
    mesh = plsc.VectorSubcoreMesh(core_axis_name="c", subcore_axis_name="s")

    @functools.partial(
        pl.kernel,
        mesh=mesh,
        compiler_params=pltpu.CompilerParams(needs_layout_passes=False),
        out_type=(
            jax.ShapeDtypeStruct((DU, B), jnp.float32),
            jax.ShapeDtypeStruct((DU, B), jnp.float32),
            jax.ShapeDtypeStruct((DC, B), jnp.float32),
        ),
        scratch_types=[
            pltpu.VMEM((B,), jnp.int32),
            pltpu.VMEM((B,), jnp.int32),
            pltpu.VMEM((B,), jnp.int32),
            pltpu.VMEM((VU,), jnp.float32),
            pltpu.VMEM((VC,), jnp.float32),
            pltpu.VMEM((B,), jnp.float32),
            pltpu.VMEM((B,), jnp.float32),
            pltpu.SemaphoreType.DMA,
            pltpu.SemaphoreType.DMA,
            pltpu.SemaphoreType.DMA,
        ],
    )
    def k(uid_h, iid_h, cid_h, wu_h, wi_h, wc_h, eu_h, ei_h, ec_h,
          uidx, iidx, cidx, rowbuf, catbuf, outA, outB,
          sem_row, sem_out, sem_pre):
        w = lax.axis_index("s") * NC + lax.axis_index("c")
        tasks = [(wu_h, w, uidx, eu_h), (wu_h, w + NW, uidx, eu_h),
                 (wi_h, w, iidx, ei_h), (wi_h, w + NW, iidx, ei_h)]
        outs = [outA, outB]

        row_copy = pltpu.async_copy(wu_h.at[w], rowbuf, sem_row)
        pre = [pltpu.async_copy(uid_h, uidx, sem_pre),
               pltpu.async_copy(iid_h, iidx, sem_pre),
               pltpu.async_copy(cid_h, cidx, sem_pre),
               pltpu.async_copy(wc_h.at[w], catbuf, sem_pre)]
        for c in pre:
            c.wait()

        def gather(idxbuf, buf, ob):
            def body(i, carry):
                for u in range(UNROLL):
                    off = (i * UNROLL + u) * 16
                    iv = idxbuf[pl.ds(off, 16)]
                    ob[pl.ds(off, 16)] = plsc.load_gather(buf, [iv])
                return carry

            lax.fori_loop(0, n_iter, body, 0)

        gather(cidx, catbuf, outB)
        out_copies = {-1: pltpu.async_copy(outB, ec_h.at[w], sem_out)}
        for t in range(4):
            row_copy.wait()
            ob = outs[t % 2]
            if t - 2 in out_copies:
                out_copies.pop(t - 2).wait()
            gather(tasks[t][2], rowbuf, ob)
            if t < 3:
                tbl2, row2, _, _ = tasks[t + 1]
                row_copy = pltpu.async_copy(tbl2.at[row2], rowbuf, sem_row)
            out_copies[t] = pltpu.async_copy(
                ob, tasks[t][3].at[tasks[t][1]], sem_out)
        out_copies.pop(2).wait()
        out_copies.pop(3).wait()

    return k(user_id, item_id, category, wuT, wiT, wcT)


def _tc_project(base, euT, eiT, ecT, W_proj, b_proj):
    B, DB = base.shape
    DU = euT.shape[0]
    DI = eiT.shape[0]
    DC = ecT.shape[0]
    N = W_proj.shape[1]
    K = W_proj.shape[0]
    BLK = 2048
    grid = (B // BLK,)
    dn_t = (((0,), (0,)), ((), ()))

    def body(base_ref, eu_ref, ei_ref, ec_ref, wp_ref, b_ref, out_ref):
        acc = jnp.dot(base_ref[...], wp_ref[0:DB, :],
                      preferred_element_type=jnp.float32)
        acc += lax.dot_general(eu_ref[...], wp_ref[DB:DB + DU, :], dn_t,
                               preferred_element_type=jnp.float32)
        acc += lax.dot_general(ei_ref[...], wp_ref[DB + DU:DB + DU + DI, :],
                               dn_t, preferred_element_type=jnp.float32)
        acc += lax.dot_general(ec_ref[...], wp_ref[DB + DU + DI:K, :], dn_t,
                               preferred_element_type=jnp.float32)
        out_ref[...] = acc + b_ref[...]

    return pl.pallas_call(
        body,
        grid=grid,
        in_specs=[
            pl.BlockSpec((BLK, DB), lambda i: (i, 0)),
            pl.BlockSpec((DU, BLK), lambda i: (0, i)),
            pl.BlockSpec((DI, BLK), lambda i: (0, i)),
            pl.BlockSpec((DC, BLK), lambda i: (0, i)),
            pl.BlockSpec((K, N), lambda i: (0, 0)),
            pl.BlockSpec((1, N), lambda i: (0, 0)),
        ],
        out_specs=pl.BlockSpec((BLK, N), lambda i: (i, 0)),
        out_shape=jax.ShapeDtypeStruct((B, N), jnp.float32),
    )(base, euT, eiT, ecT, W_proj, b_proj.reshape(1, N))


def kernel(base, user_id, item_id, category, W_user, W_item, W_cat, W_proj, b_proj):
    user_id = user_id.astype(jnp.int32)
    item_id = item_id.astype(jnp.int32)
    category = category.astype(jnp.int32)
    euT, eiT, ecT = _sc_gather_t(user_id, item_id, category,
                                 W_user.T, W_item.T, W_cat.T)
    return _tc_project(base, euT, eiT, ecT, W_proj, b_proj)


# speedup vs baseline: 1.0161x; 1.0004x over previous
import functools

import jax
import jax.numpy as jnp
from jax import lax
from jax.experimental import pallas as pl
from jax.experimental.pallas import tpu as pltpu
from jax.experimental.pallas import tpu_sc as plsc


def _sc_gather_t(user_id, item_id, category, wuT, wiT, wcT):
    info = plsc.get_sparse_core_info()
    NC, NS = info.num_cores, info.num_subcores
    NW = NC * NS
    B = user_id.shape[0]
    DU, VU = wuT.shape
    DC, VC = wcT.shape
    assert DU == 2 * NW and DC == NW
    UNROLL = 8
    n_iter = B // (16 * UNROLL)
    mesh = plsc.VectorSubcoreMesh(core_axis_name="c", subcore_axis_name="s")

    @functools.partial(
        pl.kernel,
        mesh=mesh,
        compiler_params=pltpu.CompilerParams(needs_layout_passes=False),
        out_type=(
            jax.ShapeDtypeStruct((DU, B), jnp.float32),
            jax.ShapeDtypeStruct((DU, B), jnp.float32),
            jax.ShapeDtypeStruct((DC, B), jnp.float32),
        ),
        scratch_types=[
            pltpu.VMEM((B,), jnp.int32),
            pltpu.VMEM((B,), jnp.int32),
            pltpu.VMEM((B,), jnp.int32),
            pltpu.VMEM((VU,), jnp.float32),
            pltpu.VMEM((VC,), jnp.float32),
            pltpu.VMEM((B,), jnp.float32),
            pltpu.VMEM((B,), jnp.float32),
            pltpu.SemaphoreType.DMA,
            pltpu.SemaphoreType.DMA,
            pltpu.SemaphoreType.DMA,
            pltpu.SemaphoreType.DMA,
        ],
    )
    def k(uid_h, iid_h, cid_h, wu_h, wi_h, wc_h, eu_h, ei_h, ec_h,
          uidx, iidx, cidx, rowbuf, catbuf, outA, outB,
          sem_row, sem_outA, sem_outB, sem_pre):
        w = lax.axis_index("s") * NC + lax.axis_index("c")
        tasks = [(wu_h, w, uidx, eu_h), (wu_h, w + NW, uidx, eu_h),
                 (wi_h, w, iidx, ei_h), (wi_h, w + NW, iidx, ei_h)]
        outs = [outA, outB]
        out_sems = [sem_outA, sem_outB]

        row_copy = pltpu.async_copy(wu_h.at[w], rowbuf, sem_row)
        pre = [pltpu.async_copy(uid_h, uidx, sem_pre),
               pltpu.async_copy(iid_h, iidx, sem_pre),
               pltpu.async_copy(cid_h, cidx, sem_pre),
               pltpu.async_copy(wc_h.at[w], catbuf, sem_pre)]
        for c in pre:
            c.wait()

        def gather(idxbuf, buf, ob):
            def body(i, carry):
                for u in range(UNROLL):
                    off = (i * UNROLL + u) * 16
                    iv = idxbuf[pl.ds(off, 16)]
                    ob[pl.ds(off, 16)] = plsc.load_gather(buf, [iv])
                return carry

            lax.fori_loop(0, n_iter, body, 0)

        gather(cidx, catbuf, outB)
        out_copies = {-1: pltpu.async_copy(outB, ec_h.at[w], sem_outB)}
        for t in range(4):
            row_copy.wait()
            ob = outs[t % 2]
            if t - 2 in out_copies:
                out_copies.pop(t - 2).wait()
            gather(tasks[t][2], rowbuf, ob)
            if t < 3:
                tbl2, row2, _, _ = tasks[t + 1]
                row_copy = pltpu.async_copy(tbl2.at[row2], rowbuf, sem_row)
            out_copies[t] = pltpu.async_copy(
                ob, tasks[t][3].at[tasks[t][1]], out_sems[t % 2])
        out_copies.pop(2).wait()
        out_copies.pop(3).wait()

    return k(user_id, item_id, category, wuT, wiT, wcT)


def _tc_project(base, euT, eiT, ecT, W_proj, b_proj):
    B, DB = base.shape
    DU = euT.shape[0]
    DI = eiT.shape[0]
    DC = ecT.shape[0]
    N = W_proj.shape[1]
    K = W_proj.shape[0]
    BLK = 2048
    grid = (B // BLK,)
    dn_t = (((0,), (0,)), ((), ()))

    def body(base_ref, eu_ref, ei_ref, ec_ref, wp_ref, b_ref, out_ref):
        acc = jnp.dot(base_ref[...], wp_ref[0:DB, :],
                      preferred_element_type=jnp.float32)
        acc += lax.dot_general(eu_ref[...], wp_ref[DB:DB + DU, :], dn_t,
                               preferred_element_type=jnp.float32)
        acc += lax.dot_general(ei_ref[...], wp_ref[DB + DU:DB + DU + DI, :],
                               dn_t, preferred_element_type=jnp.float32)
        acc += lax.dot_general(ec_ref[...], wp_ref[DB + DU + DI:K, :], dn_t,
                               preferred_element_type=jnp.float32)
        out_ref[...] = acc + b_ref[...]

    return pl.pallas_call(
        body,
        grid=grid,
        in_specs=[
            pl.BlockSpec((BLK, DB), lambda i: (i, 0)),
            pl.BlockSpec((DU, BLK), lambda i: (0, i)),
            pl.BlockSpec((DI, BLK), lambda i: (0, i)),
            pl.BlockSpec((DC, BLK), lambda i: (0, i)),
            pl.BlockSpec((K, N), lambda i: (0, 0)),
            pl.BlockSpec((1, N), lambda i: (0, 0)),
        ],
        out_specs=pl.BlockSpec((BLK, N), lambda i: (i, 0)),
        out_shape=jax.ShapeDtypeStruct((B, N), jnp.float32),
    )(base, euT, eiT, ecT, W_proj, b_proj.reshape(1, N))


def kernel(base, user_id, item_id, category, W_user, W_item, W_cat, W_proj, b_proj):
    user_id = user_id.astype(jnp.int32)
    item_id = item_id.astype(jnp.int32)
    category = category.astype(jnp.int32)
    euT, eiT, ecT = _sc_gather_t(user_id, item_id, category,
                                 W_user.T, W_item.T, W_cat.T)
    return _tc_project(base, euT, eiT, ecT, W_proj, b_proj)
